# Initial kernel scaffold; baseline (speedup 1.0000x reference)
import jax
import jax.numpy as jnp
from jax.experimental import pallas as pl


def kernel(adapter_feats, v2p_map, caption_embed, point_idx, segment_ids, logit_scale_param):
    raise NotImplementedError("write your pallas kernel here")


# speedup vs baseline: 4.1842x; 4.1842x over previous
import functools

import jax
import jax.numpy as jnp
from jax import lax
from jax.experimental import pallas as pl
from jax.experimental.pallas import tpu as pltpu, tpu_sc as plsc

NV = 50000
D = 256
C = 512
NP = 100000
NSEG = 5000
NSEG_PAD = 5120
ROWS_PER_TILE = 320
NW = 32
BATCH = 112
NBATCH = 28
CHUNK = BATCH * NBATCH
NP_PAD = NW * CHUNK

RB = 1000
SB = 1000


def _stage_b(adapter_feats, caption_embed, param11):
    def body(p_ref, f_ref, e_ref, nf_ref, ax_ref):
        x = f_ref[...]
        s2 = jnp.sum(x * x, axis=1, keepdims=True)
        n = x / (jnp.sqrt(s2) + 1e-12)
        nf_ref[...] = n
        scale = jnp.exp(p_ref[0, 0])
        logits = lax.dot_general(
            n, e_ref[...], (((1,), (1,)), ((), ())),
            preferred_element_type=jnp.float32) * scale
        m = jnp.max(logits, axis=1, keepdims=True)
        lse = m + jnp.log(jnp.sum(jnp.exp(logits - m), axis=1, keepdims=True))
        iot = lax.broadcasted_iota(jnp.int32, (RB, 16), 1)
        ax_ref[...] = jnp.where(iot == 0, lse,
                                jnp.where(iot == 1, jnp.float32(1.0),
                                          jnp.float32(0.0)))

    return pl.pallas_call(
        body,
        grid=(NV // RB,),
        in_specs=[
            pl.BlockSpec((1, 1), lambda i: (i * 0, i * 0), memory_space=pltpu.SMEM),
            pl.BlockSpec((RB, D), lambda i: (i, i * 0)),
            pl.BlockSpec((C, D), lambda i: (i * 0, i * 0)),
        ],
        out_specs=[
            pl.BlockSpec((RB, D), lambda i: (i, i * 0)),
            pl.BlockSpec((RB, 16), lambda i: (i, i * 0)),
        ],
        out_shape=[
            jax.ShapeDtypeStruct((NV, D), jnp.float32),
            jax.ShapeDtypeStruct((NV, 16), jnp.float32),
        ],
    )(param11, adapter_feats, caption_embed)


def _stage_c(nf, ax, v2p_i32, pt3, seg3, zg, za):
    mesh = plsc.VectorSubcoreMesh(
        core_axis_name="c", subcore_axis_name="s", num_cores=2, num_subcores=16)

    @functools.partial(
        pl.kernel,
        compiler_params=pltpu.CompilerParams(use_tc_tiling_on_sc=False),
        out_type=(
            jax.ShapeDtypeStruct((2, NSEG_PAD, D), jnp.float32),
            jax.ShapeDtypeStruct((2, NSEG_PAD, 16), jnp.float32),
        ),
        mesh=mesh,
        scratch_types=[
            pltpu.VMEM((NBATCH, BATCH), jnp.int32),
            pltpu.VMEM((NBATCH, BATCH), jnp.int32),
            pltpu.VMEM((BATCH,), jnp.int32),
            pltpu.VMEM((BATCH,), jnp.int32),
            pltpu.VMEM((BATCH, D), jnp.float32),
            pltpu.VMEM((BATCH, 16), jnp.float32),
            pltpu.VMEM_SHARED((NSEG_PAD, D), jnp.float32),
            pltpu.VMEM_SHARED((NSEG_PAD, 16), jnp.float32),
            pltpu.SemaphoreType.DMA,
        ],
    )
    def sc_call(nf_hbm, ax_hbm, v2p_hbm, pt_hbm, seg_hbm, zg_hbm, za_hbm,
                g_out, a_out, pt_v, seg_v, u_v, seg_b, rows_v, arow_v,
                g_sh, a_sh, sem):
        cid = lax.axis_index("c")
        sid = lax.axis_index("s")
        wid = cid * 16 + sid
        pltpu.sync_copy(zg_hbm, g_sh.at[pl.ds(sid * ROWS_PER_TILE, ROWS_PER_TILE)])
        pltpu.sync_copy(za_hbm, a_sh.at[pl.ds(sid * ROWS_PER_TILE, ROWS_PER_TILE)])
        pltpu.sync_copy(pt_hbm.at[wid], pt_v)
        pltpu.sync_copy(seg_hbm.at[wid], seg_v)
        plsc.subcore_barrier()

        def batch(j, carry):
            for k in range(BATCH // 16):
                sl = pl.ds(k * 16, 16)
                seg_b[sl] = seg_v[j, sl]
            pltpu.async_copy(v2p_hbm.at[pt_v.at[j]], u_v, sem).wait()
            pltpu.async_copy(nf_hbm.at[u_v], rows_v, sem).wait()
            pltpu.async_copy(ax_hbm.at[u_v], arow_v, sem).wait()
            pltpu.sync_copy(rows_v, g_sh.at[seg_b], add=True)
            pltpu.sync_copy(arow_v, a_sh.at[seg_b], add=True)
            return carry

        lax.fori_loop(0, NBATCH, batch, 0)
        plsc.subcore_barrier()
        rows = pl.ds(sid * ROWS_PER_TILE, ROWS_PER_TILE)
        pltpu.sync_copy(g_sh.at[rows], g_out.at[cid, rows])
        pltpu.sync_copy(a_sh.at[rows], a_out.at[cid, rows])

    return sc_call(nf, ax, v2p_i32, pt3, seg3, zg, za)


def _stage_d(g0, g1, a0, a1, caption_embed, param11):
    def body(p_ref, g0_ref, g1_ref, a0_ref, a1_ref, e_ref, out_ref, cnt_ref):
        g = g0_ref[...] + g1_ref[...]
        a = a0_ref[...] + a1_ref[...]
        lsum = a[:, 0:1]
        cnt = a[:, 1:2]
        scale = jnp.exp(p_ref[0, 0])
        logits = lax.dot_general(
            g, e_ref[...], (((1,), (1,)), ((), ())),
            preferred_element_type=jnp.float32) * scale
        denom = jnp.where(cnt > 0, 1.0 / jnp.maximum(cnt, 1.0),
                          jnp.float32(0.0))
        out_ref[...] = (logits - lsum) * denom
        cnt_ref[...] = jnp.broadcast_to(cnt, (SB, 8))

    return pl.pallas_call(
        body,
        grid=(NSEG // SB,),
        in_specs=[
            pl.BlockSpec((1, 1), lambda i: (i * 0, i * 0), memory_space=pltpu.SMEM),
            pl.BlockSpec((SB, D), lambda i: (i, i * 0)),
            pl.BlockSpec((SB, D), lambda i: (i, i * 0)),
            pl.BlockSpec((SB, 16), lambda i: (i, i * 0)),
            pl.BlockSpec((SB, 16), lambda i: (i, i * 0)),
            pl.BlockSpec((C, D), lambda i: (i * 0, i * 0)),
        ],
        out_specs=[
            pl.BlockSpec((SB, C), lambda i: (i, i * 0)),
            pl.BlockSpec((SB, 8), lambda i: (i, i * 0)),
        ],
        out_shape=[
            jax.ShapeDtypeStruct((NSEG, C), jnp.float32),
            jax.ShapeDtypeStruct((NSEG, 8), jnp.float32),
        ],
    )(param11, g0, g1, a0, a1, caption_embed)


def kernel(adapter_feats, v2p_map, caption_embed, point_idx, segment_ids,
           logit_scale_param):
    adapter_feats = adapter_feats.astype(jnp.float32)
    caption_embed = caption_embed.astype(jnp.float32)
    param11 = jnp.reshape(logit_scale_param.astype(jnp.float32), (1, 1))

    v2p_i32 = v2p_map.astype(jnp.int32)
    pt = jnp.pad(point_idx.astype(jnp.int32), (0, NP_PAD - NP))
    seg = jnp.pad(segment_ids.astype(jnp.int32), (0, NP_PAD - NP),
                  constant_values=NSEG)
    pt3 = jnp.reshape(pt, (NW, NBATCH, BATCH))
    seg3 = jnp.reshape(seg, (NW, NBATCH, BATCH))
    zg = jnp.zeros((ROWS_PER_TILE, D), jnp.float32)
    za = jnp.zeros((ROWS_PER_TILE, 16), jnp.float32)

    nf, ax = _stage_b(adapter_feats, caption_embed, param11)
    g_out, a_out = _stage_c(nf, ax, v2p_i32, pt3, seg3, zg, za)
    pooled, cnt8 = _stage_d(g_out[0, :NSEG], g_out[1, :NSEG],
                            a_out[0, :NSEG], a_out[1, :NSEG],
                            caption_embed, param11)
    zero_loss = jnp.zeros((), jnp.float32)
    return pooled, cnt8[:, 0], zero_loss
